# Initial kernel scaffold; baseline (speedup 1.0000x reference)
import jax
import jax.numpy as jnp
from jax.experimental import pallas as pl


def kernel(x, w1, b1, w2, b2, wd, bd, wu, bu, wf, bf):
    raise NotImplementedError("write your pallas kernel here")


# speedup vs baseline: 1.4439x; 1.4439x over previous
import functools

import jax
import jax.numpy as jnp
from jax.experimental import pallas as pl
from jax.experimental.pallas import tpu as pltpu


def _rcag_kernel(x_ref, w1_ref, b1_ref, w2_ref, b2_ref,
                 wd_ref, bd_ref, wu_ref, bu_ref,
                 wf_ref, bf_ref, mask_ref,
                 out_ref, *, H, W, C, nb):
    HW = H * W
    x = x_ref[0]

    taps = []
    mi = 0
    for dy in (-1, 0, 1):
        for dx in (-1, 0, 1):
            if dy == 0 and dx == 0:
                continue
            taps.append(((dy + 1) * 3 + (dx + 1), dy * W + dx, mi))
            mi += 1

    def conv3x3(a_bf, w_ref, blk, b):
        acc = jnp.dot(w_ref[blk, 4], a_bf, preferred_element_type=jnp.float32)
        for t, sh, m in taps:
            tap = pltpu.roll(a_bf, (-sh) % HW, 1) * mask_ref[m]
            acc = acc + jnp.dot(w_ref[blk, t], tap,
                                preferred_element_type=jnp.float32)
        return acc + b

    a = x
    for blk in range(nb):
        a_bf = a.astype(jnp.bfloat16)
        r = jnp.maximum(conv3x3(a_bf, w1_ref, blk, b1_ref[blk]), 0.0)
        r = conv3x3(r.astype(jnp.bfloat16), w2_ref, blk, b2_ref[blk])
        y = jnp.sum(r, axis=1, keepdims=True) * (1.0 / HW)
        d = jnp.maximum(jnp.sum(wd_ref[blk] * y, axis=0, keepdims=True)
                        + bd_ref[blk], 0.0)
        s = jax.nn.sigmoid(jnp.sum(wu_ref[blk] * d, axis=1, keepdims=True)
                           + bu_ref[blk])
        a = r * s + a

    res = conv3x3(a.astype(jnp.bfloat16), wf_ref, 0, bf_ref[...])
    out_ref[0] = (res + x).astype(out_ref.dtype)


def kernel(x, w1, b1, w2, b2, wd, bd, wu, bu, wf, bf):
    N, C, H, W = x.shape
    HW = H * W
    nb = w1.shape[0]
    Cr = wd.shape[-1]

    w1b = w1.astype(jnp.bfloat16)
    w2b = w2.astype(jnp.bfloat16)
    wfb = wf.astype(jnp.bfloat16)

    row = jnp.arange(HW, dtype=jnp.int32) // W
    col = jnp.arange(HW, dtype=jnp.int32) % W
    ms = []
    for dy in (-1, 0, 1):
        for dx in (-1, 0, 1):
            if dy == 0 and dx == 0:
                continue
            valid = jnp.ones((HW,), jnp.bool_)
            if dy == -1:
                valid = valid & (row != 0)
            if dy == 1:
                valid = valid & (row != H - 1)
            if dx == -1:
                valid = valid & (col != 0)
            if dx == 1:
                valid = valid & (col != W - 1)
            ms.append(valid.astype(jnp.bfloat16))
    masks = jnp.stack(ms).reshape(8, 1, HW)

    kernel_fn = functools.partial(_rcag_kernel, H=H, W=W, C=C, nb=nb)

    def full(shape):
        return pl.BlockSpec(shape, lambda n, _s=shape: (0,) * len(_s))

    out = pl.pallas_call(
        kernel_fn,
        out_shape=jax.ShapeDtypeStruct((N, C, HW), x.dtype),
        grid_spec=pltpu.PrefetchScalarGridSpec(
            num_scalar_prefetch=0,
            grid=(N,),
            in_specs=[
                pl.BlockSpec((1, C, HW), lambda n: (n, 0, 0)),
                full((nb, 9, C, C)), full((nb, C, 1)),
                full((nb, 9, C, C)), full((nb, C, 1)),
                full((nb, C, Cr)), full((nb, 1, Cr)),
                full((nb, C, Cr)), full((nb, C, 1)),
                full((1, 9, C, C)), full((C, 1)),
                full((8, 1, HW)),
            ],
            out_specs=pl.BlockSpec((1, C, HW), lambda n: (n, 0, 0)),
        ),
        compiler_params=pltpu.CompilerParams(dimension_semantics=("parallel",)),
    )(x.reshape(N, C, HW),
      w1b, b1, w2b, b2, wd, bd, wu, bu, wfb, bf, masks)
    return out.reshape(N, C, H, W)


# speedup vs baseline: 1.6360x; 1.1330x over previous
import functools

import jax
import jax.numpy as jnp
from jax.experimental import pallas as pl
from jax.experimental.pallas import tpu as pltpu


def _rcag_kernel(x_ref, w1_ref, b1_ref, w2_ref, b2_ref,
                 wd_ref, bd_ref, wu_ref, bu_ref,
                 wf_ref, bf_ref, mrow_ref, mcol_ref,
                 out_ref, *, H, W, C, nb):
    HW = H * W
    x = x_ref[0]

    def conv3x3(a_bf, w_ref, blk, b):
        aU = pltpu.roll(a_bf, W, 1) * mrow_ref[0]
        aD = pltpu.roll(a_bf, HW - W, 1) * mrow_ref[1]
        stack = jnp.concatenate([aU, a_bf, aD], axis=0)
        B = jnp.dot(w_ref[blk], stack, preferred_element_type=jnp.float32)
        Bm, B0, Bp = B[0:C], B[C:2 * C], B[2 * C:3 * C]
        return (B0 + b
                + pltpu.roll(Bm, 1, 1) * mcol_ref[0]
                + pltpu.roll(Bp, HW - 1, 1) * mcol_ref[1])

    a = x
    for blk in range(nb):
        a_bf = a.astype(jnp.bfloat16)
        r = jnp.maximum(conv3x3(a_bf, w1_ref, blk, b1_ref[blk]), 0.0)
        r = conv3x3(r.astype(jnp.bfloat16), w2_ref, blk, b2_ref[blk])
        y = jnp.sum(r, axis=1, keepdims=True) * (1.0 / HW)
        d = jnp.maximum(jnp.sum(wd_ref[blk] * y, axis=0, keepdims=True)
                        + bd_ref[blk], 0.0)
        s = jax.nn.sigmoid(jnp.sum(wu_ref[blk] * d, axis=1, keepdims=True)
                           + bu_ref[blk])
        a = r * s + a

    res = conv3x3(a.astype(jnp.bfloat16), wf_ref, 0, bf_ref[...])
    out_ref[0] = (res + x).astype(out_ref.dtype)


def _stack_weights(w, C):
    nb = w.shape[0]
    return jnp.transpose(w.reshape(nb, 3, 3, C, C),
                         (0, 2, 3, 1, 4)).reshape(nb, 3 * C, 3 * C)


def kernel(x, w1, b1, w2, b2, wd, bd, wu, bu, wf, bf):
    N, C, H, W = x.shape
    HW = H * W
    nb = w1.shape[0]
    Cr = wd.shape[-1]

    w1s = _stack_weights(w1, C).astype(jnp.bfloat16)
    w2s = _stack_weights(w2, C).astype(jnp.bfloat16)
    wfs = _stack_weights(wf, C).astype(jnp.bfloat16)

    row = jnp.arange(HW, dtype=jnp.int32) // W
    col = jnp.arange(HW, dtype=jnp.int32) % W
    mrow = jnp.stack([(row != 0).astype(jnp.bfloat16),
                      (row != H - 1).astype(jnp.bfloat16)]).reshape(2, 1, HW)
    mcol = jnp.stack([(col != 0).astype(jnp.float32),
                      (col != W - 1).astype(jnp.float32)]).reshape(2, 1, HW)

    kernel_fn = functools.partial(_rcag_kernel, H=H, W=W, C=C, nb=nb)

    def full(shape):
        return pl.BlockSpec(shape, lambda n, _s=shape: (0,) * len(_s))

    out = pl.pallas_call(
        kernel_fn,
        out_shape=jax.ShapeDtypeStruct((N, C, HW), x.dtype),
        grid_spec=pltpu.PrefetchScalarGridSpec(
            num_scalar_prefetch=0,
            grid=(N,),
            in_specs=[
                pl.BlockSpec((1, C, HW), lambda n: (n, 0, 0)),
                full((nb, 3 * C, 3 * C)), full((nb, C, 1)),
                full((nb, 3 * C, 3 * C)), full((nb, C, 1)),
                full((nb, C, Cr)), full((nb, 1, Cr)),
                full((nb, C, Cr)), full((nb, C, 1)),
                full((1, 3 * C, 3 * C)), full((C, 1)),
                full((2, 1, HW)), full((2, 1, HW)),
            ],
            out_specs=pl.BlockSpec((1, C, HW), lambda n: (n, 0, 0)),
        ),
        compiler_params=pltpu.CompilerParams(dimension_semantics=("parallel",)),
    )(x.reshape(N, C, HW),
      w1s, b1, w2s, b2, wd, bd, wu, bu, wfs, bf, mrow, mcol)
    return out.reshape(N, C, H, W)


# speedup vs baseline: 1.6420x; 1.0037x over previous
import functools

import jax
import jax.numpy as jnp
from jax.experimental import pallas as pl
from jax.experimental.pallas import tpu as pltpu


def _rcag_kernel(x_ref, w1_ref, w2_ref, wd_ref, bd_ref, wu_ref, bu_ref,
                 wf_ref, mcol_ref, out_ref, s_ref, *, H, W, C, nb):
    HW = H * W
    x = x_ref[0]

    s_ref[0:C, 0:W] = jnp.zeros((C, W), jnp.bfloat16)
    s_ref[2 * C:3 * C, pl.ds(HW - W, W)] = jnp.zeros((C, W), jnp.bfloat16)
    pad = (jax.lax.broadcasted_iota(jnp.int32, (8, HW), 0) == 0
           ).astype(jnp.bfloat16)
    s_ref[3 * C:3 * C + 8, :] = pad

    def conv3x3(a_bf, w_ref, blk):
        s_ref[0:C, pl.ds(W, HW - W)] = a_bf[:, :HW - W]
        s_ref[C:2 * C, :] = a_bf
        s_ref[2 * C:3 * C, 0:HW - W] = a_bf[:, W:]
        B = jnp.dot(w_ref[blk], s_ref[:], preferred_element_type=jnp.float32)
        return (B[C:2 * C]
                + pltpu.roll(B[0:C], 1, 1) * mcol_ref[0]
                + pltpu.roll(B[2 * C:3 * C], HW - 1, 1) * mcol_ref[1])

    a = x
    for blk in range(nb):
        r = jnp.maximum(conv3x3(a.astype(jnp.bfloat16), w1_ref, blk), 0.0)
        r = conv3x3(r.astype(jnp.bfloat16), w2_ref, blk)
        y = jnp.sum(r, axis=1, keepdims=True) * (1.0 / HW)
        d = jnp.maximum(jnp.sum(wd_ref[blk] * y, axis=0, keepdims=True)
                        + bd_ref[blk], 0.0)
        s = jax.nn.sigmoid(jnp.sum(wu_ref[blk] * d, axis=1, keepdims=True)
                           + bu_ref[blk])
        a = r * s + a

    res = conv3x3(a.astype(jnp.bfloat16), wf_ref, 0)
    out_ref[0] = (res + x).astype(out_ref.dtype)


def _stack_weights(w, b, C):
    nb = w.shape[0]
    base = jnp.transpose(w.reshape(nb, 3, 3, C, C),
                         (0, 2, 3, 1, 4)).reshape(nb, 3 * C, 3 * C)
    extra = jnp.zeros((nb, 3 * C, 8), w.dtype)
    extra = extra.at[:, C:2 * C, 0].set(b.reshape(nb, C))
    return jnp.concatenate([base, extra], axis=2).astype(jnp.bfloat16)


def kernel(x, w1, b1, w2, b2, wd, bd, wu, bu, wf, bf):
    N, C, H, W = x.shape
    HW = H * W
    nb = w1.shape[0]
    Cr = wd.shape[-1]

    w1s = _stack_weights(w1, b1, C)
    w2s = _stack_weights(w2, b2, C)
    wfs = _stack_weights(wf, bf.reshape(1, C, 1), C)

    col = jnp.arange(HW, dtype=jnp.int32) % W
    mcol = jnp.stack([(col != 0).astype(jnp.float32),
                      (col != W - 1).astype(jnp.float32)]).reshape(2, 1, HW)

    kernel_fn = functools.partial(_rcag_kernel, H=H, W=W, C=C, nb=nb)

    def full(shape):
        return pl.BlockSpec(shape, lambda n, _s=shape: (0,) * len(_s))

    out = pl.pallas_call(
        kernel_fn,
        out_shape=jax.ShapeDtypeStruct((N, C, HW), x.dtype),
        grid_spec=pltpu.PrefetchScalarGridSpec(
            num_scalar_prefetch=0,
            grid=(N,),
            in_specs=[
                pl.BlockSpec((1, C, HW), lambda n: (n, 0, 0)),
                full((nb, 3 * C, 3 * C + 8)),
                full((nb, 3 * C, 3 * C + 8)),
                full((nb, C, Cr)), full((nb, 1, Cr)),
                full((nb, C, Cr)), full((nb, C, 1)),
                full((1, 3 * C, 3 * C + 8)),
                full((2, 1, HW)),
            ],
            out_specs=pl.BlockSpec((1, C, HW), lambda n: (n, 0, 0)),
            scratch_shapes=[pltpu.VMEM((3 * C + 8, HW), jnp.bfloat16)],
        ),
        compiler_params=pltpu.CompilerParams(dimension_semantics=("parallel",)),
    )(x.reshape(N, C, HW),
      w1s, w2s, wd, bd, wu, bu, wfs, mcol)
    return out.reshape(N, C, H, W)
